# baseline (speedup 1.0000x reference)
import functools

import jax
import jax.numpy as jnp
from jax import lax
from jax.experimental import pallas as pl
from jax.experimental.pallas import tpu as pltpu
from jax.experimental.pallas import tpu_sc as plsc

N_TABLES = 26
SPLIT = 17
DIM = 32
N_OUT = 5
NC, NS = 2, 16
NW = NC * NS
VPAD = 100352
CHUNK = 8192


def _proj_body(w_ref, t_ref, out_ref):
    out_ref[0] = jax.lax.dot_general(
        w_ref[0],
        t_ref[0],
        (((1,), (0,)), ((), ())),
        preferred_element_type=jnp.float32,
    )


def _tc_project(w8, tables_t, j0, nj):
    dim = tables_t.shape[1]
    blk = VPAD // 2
    return pl.pallas_call(
        _proj_body,
        grid=(nj, 2),
        in_specs=[
            pl.BlockSpec((1, 8, dim), lambda j, c: (j0 + j, 0, 0)),
            pl.BlockSpec((1, dim, blk), lambda j, c: (j0 + j, 0, c)),
        ],
        out_specs=pl.BlockSpec((1, 8, blk), lambda j, c: (j, 0, c)),
        out_shape=jax.ShapeDtypeStruct((nj, 8, VPAD), jnp.float32),
    )(w8, tables_t)


def _lookup_body(j0, nj, idx_hbm, p_hbm, val_hbm, row_v, idx_v, val_v, sem):
    wid = lax.axis_index("s") * NC + lax.axis_index("c")
    batch = idx_hbm.shape[1]
    n_chunks = batch // CHUNK
    n_tasks = nj * N_OUT
    rounds = -(-n_tasks // NW)

    @pl.loop(0, rounds)
    def _task_loop(s):
        t = s * NW + wid

        @pl.when(t < n_tasks)
        def _():
            j = t // N_OUT
            o = lax.rem(t, N_OUT)
            pltpu.sync_copy(p_hbm.at[j, o], row_v)

            @pl.loop(0, n_chunks)
            def _chunk(c):
                pltpu.sync_copy(
                    idx_hbm.at[j0 + j, pl.ds(c * CHUNK, CHUNK)], idx_v
                )

                @pl.loop(0, CHUNK // 16)
                def _group(g):
                    iv = idx_v[pl.ds(g * 16, 16)]
                    val_v[pl.ds(g * 16, 16)] = plsc.load_gather(row_v, [iv])

                pltpu.sync_copy(
                    val_v, val_hbm.at[j, o, pl.ds(c * CHUNK, CHUNK)]
                )


def _sc_lookup(inputs, p, j0, nj):
    batch = inputs.shape[1]
    mesh = plsc.VectorSubcoreMesh(core_axis_name="c", subcore_axis_name="s")
    return pl.kernel(
        functools.partial(_lookup_body, j0, nj),
        out_type=jax.ShapeDtypeStruct((nj, 8, batch), jnp.float32),
        mesh=mesh,
        scratch_types=[
            pltpu.VMEM((VPAD,), jnp.float32),
            pltpu.VMEM((CHUNK,), jnp.int32),
            pltpu.VMEM((CHUNK,), jnp.float32),
            pltpu.SemaphoreType.DMA,
        ],
        compiler_params=pltpu.CompilerParams(
            use_tc_tiling_on_sc=False, needs_layout_passes=False
        ),
    )(inputs, p)


def _reduce_body(va_ref, vb_ref, b_ref, out_ref):
    acc = jnp.zeros(va_ref.shape[1:], dtype=jnp.float32)
    for j in range(va_ref.shape[0]):
        acc = acc + va_ref[j]
    for j in range(vb_ref.shape[0]):
        acc = acc + vb_ref[j]
    out_ref[...] = acc[:N_OUT, :].T + b_ref[...]


def _tc_reduce(val_a, val_b, b2d):
    batch = val_a.shape[2]
    blk = 4096
    na, nb = val_a.shape[0], val_b.shape[0]
    return pl.pallas_call(
        _reduce_body,
        grid=(batch // blk,),
        in_specs=[
            pl.BlockSpec((na, 8, blk), lambda i: (0, 0, i)),
            pl.BlockSpec((nb, 8, blk), lambda i: (0, 0, i)),
            pl.BlockSpec((1, N_OUT), lambda i: (0, 0)),
        ],
        out_specs=pl.BlockSpec((blk, N_OUT), lambda i: (i, 0)),
        out_shape=jax.ShapeDtypeStruct((batch, N_OUT), jnp.float32),
    )(val_a, val_b, b2d)


@jax.jit
def kernel(inputs, tables, W, b):
    n, vocab, dim = tables.shape
    tables_t = jnp.transpose(tables, (0, 2, 1))
    w8 = jnp.zeros((n, 8, dim), W.dtype).at[:, :N_OUT, :].set(
        jnp.transpose(W.reshape(n, dim, N_OUT), (0, 2, 1))
    )
    p_a = _tc_project(w8, tables_t, 0, SPLIT)
    val_a = _sc_lookup(inputs, p_a, 0, SPLIT)
    p_b = _tc_project(w8, tables_t, SPLIT, N_TABLES - SPLIT)
    val_b = _sc_lookup(inputs, p_b, SPLIT, N_TABLES - SPLIT)
    return _tc_reduce(val_a, val_b, b.reshape(1, -1))


# speedup vs baseline: 1.4132x; 1.4132x over previous
import functools

import jax
import jax.numpy as jnp
from jax import lax
from jax.experimental import pallas as pl
from jax.experimental.pallas import tpu as pltpu
from jax.experimental.pallas import tpu_sc as plsc

N_TABLES = 26
SPLIT = 17
DIM = 32
N_OUT = 5
NC, NS = 2, 16
NW = NC * NS
VPAD = 100352
CHUNK = 8192


def _proj_body(w_ref, t_ref, out_ref):
    res = jax.lax.dot_general(
        w_ref[0],
        t_ref[0],
        (((1,), (0,)), ((), ())),
        preferred_element_type=jnp.float32,
    )
    res3 = res.reshape(8, res.shape[1] // 128, 128)
    out_ref[0] = jnp.transpose(res3, (1, 0, 2))


def _tc_project(w8, tables_t, j0, nj):
    dim = tables_t.shape[1]
    blk = VPAD // 2
    kb = blk // 128
    return pl.pallas_call(
        _proj_body,
        grid=(nj, 2),
        in_specs=[
            pl.BlockSpec((1, 8, dim), lambda j, c: (j0 + j, 0, 0)),
            pl.BlockSpec((1, dim, blk), lambda j, c: (j0 + j, 0, c)),
        ],
        out_specs=pl.BlockSpec((1, kb, 8, 128), lambda j, c: (j, c, 0, 0)),
        out_shape=jax.ShapeDtypeStruct((nj, VPAD // 128, 8, 128), jnp.float32),
    )(w8, tables_t)


def _lookup_body(j0, nj, idx_hbm, p_hbm, val_hbm, row_v, idx_v, val_v, sem):
    wid = lax.axis_index("s") * NC + lax.axis_index("c")
    batch = idx_hbm.shape[1]
    n_chunks = batch // CHUNK
    n_tasks = nj * N_OUT
    rounds = -(-n_tasks // NW)

    @pl.loop(0, rounds)
    def _task_loop(s):
        t = s * NW + wid

        @pl.when(t < n_tasks)
        def _():
            j = t // N_OUT
            o = lax.rem(t, N_OUT)
            pltpu.sync_copy(p_hbm.at[j, :, o, :], row_v)

            @pl.loop(0, n_chunks)
            def _chunk(c):
                pltpu.sync_copy(
                    idx_hbm.at[j0 + j, pl.ds(c * CHUNK, CHUNK)], idx_v
                )

                @pl.loop(0, CHUNK // 16)
                def _group(g):
                    iv = idx_v[pl.ds(g * 16, 16)]
                    val_v[pl.ds(g * 16, 16)] = plsc.load_gather(
                        row_v, [iv >> 7, iv & 127]
                    )

                pltpu.sync_copy(
                    val_v, val_hbm.at[j, o, pl.ds(c * CHUNK, CHUNK)]
                )


def _sc_lookup(inputs, p, j0, nj):
    batch = inputs.shape[1]
    mesh = plsc.VectorSubcoreMesh(core_axis_name="c", subcore_axis_name="s")
    return pl.kernel(
        functools.partial(_lookup_body, j0, nj),
        out_type=jax.ShapeDtypeStruct((nj, 8, batch), jnp.float32),
        mesh=mesh,
        scratch_types=[
            pltpu.VMEM((VPAD // 128, 128), jnp.float32),
            pltpu.VMEM((CHUNK,), jnp.int32),
            pltpu.VMEM((CHUNK,), jnp.float32),
            pltpu.SemaphoreType.DMA,
        ],
        compiler_params=pltpu.CompilerParams(
            use_tc_tiling_on_sc=False, needs_layout_passes=False
        ),
    )(inputs, p)


def _reduce_body(va_ref, vb_ref, b_ref, out_ref):
    acc = jnp.zeros(va_ref.shape[1:], dtype=jnp.float32)
    for j in range(va_ref.shape[0]):
        acc = acc + va_ref[j]
    for j in range(vb_ref.shape[0]):
        acc = acc + vb_ref[j]
    out_ref[...] = acc[:N_OUT, :].T + b_ref[...]


def _tc_reduce(val_a, val_b, b2d):
    batch = val_a.shape[2]
    blk = 4096
    na, nb = val_a.shape[0], val_b.shape[0]
    return pl.pallas_call(
        _reduce_body,
        grid=(batch // blk,),
        in_specs=[
            pl.BlockSpec((na, 8, blk), lambda i: (0, 0, i)),
            pl.BlockSpec((nb, 8, blk), lambda i: (0, 0, i)),
            pl.BlockSpec((1, N_OUT), lambda i: (0, 0)),
        ],
        out_specs=pl.BlockSpec((blk, N_OUT), lambda i: (i, 0)),
        out_shape=jax.ShapeDtypeStruct((batch, N_OUT), jnp.float32),
    )(val_a, val_b, b2d)


@jax.jit
def kernel(inputs, tables, W, b):
    n, vocab, dim = tables.shape
    tables_t = jnp.transpose(tables, (0, 2, 1))
    w8 = jnp.zeros((n, 8, dim), W.dtype).at[:, :N_OUT, :].set(
        jnp.transpose(W.reshape(n, dim, N_OUT), (0, 2, 1))
    )
    p_a = _tc_project(w8, tables_t, 0, SPLIT)
    val_a = _sc_lookup(inputs, p_a, 0, SPLIT)
    p_b = _tc_project(w8, tables_t, SPLIT, N_TABLES - SPLIT)
    val_b = _sc_lookup(inputs, p_b, SPLIT, N_TABLES - SPLIT)
    return _tc_reduce(val_a, val_b, b.reshape(1, -1))
